# baseline (speedup 1.0000x reference)
import functools

import jax
import jax.numpy as jnp
from jax import lax
from jax.experimental import pallas as pl
from jax.experimental.pallas import tpu as pltpu

NUM_CORR = 256
EPS = 1e-08
BIG = 3.0e38


def _tile_top2_axis0(dist, nrow):
    row = lax.broadcasted_iota(jnp.int32, dist.shape, 0)
    d0 = jnp.min(dist, axis=0)
    a0 = jnp.min(jnp.where(dist == d0[None, :], row, nrow), axis=0)
    dist_m = jnp.where(row == a0[None, :], BIG, dist)
    d1 = jnp.min(dist_m, axis=0)
    a1 = jnp.min(jnp.where(dist_m == d1[None, :], row, nrow), axis=0)
    return d0, d1, a0, a1


def _merge_top2(ad0, ad1, ai0, ai1, td0, td1, ti0, ti1):
    a_first = ad0 <= td0
    d0 = jnp.where(a_first, ad0, td0)
    i0 = jnp.where(a_first, ai0, ti0)
    loser_d = jnp.where(a_first, td0, ad0)
    loser_i = jnp.where(a_first, ti0, ai0)
    inner_a = ad1 <= td1
    inner_d = jnp.where(inner_a, ad1, td1)
    inner_i = jnp.where(inner_a, ai1, ti1)
    take_loser = loser_d <= inner_d
    d1 = jnp.where(take_loser, loser_d, inner_d)
    i1 = jnp.where(take_loser, loser_i, inner_i)
    return d0, d1, i0, i1


def _stage1_body(q_ref, st_ref, ridx_ref, cidx_ref,
                 racc_ref, ridx_acc_ref, cacc_ref, cidx_acc_ref,
                 *, bm, bn, nj, ni, m):
    i = pl.program_id(0)
    j = pl.program_id(1)
    q = q_ref[...]
    st = st_ref[...]
    dot = jax.lax.dot_general(q, st, (((1,), (0,)), ((), ())),
                              preferred_element_type=jnp.float32)
    dot_t = jax.lax.dot_general(st, q, (((0,), (1,)), ((), ())),
                                preferred_element_type=jnp.float32)
    qsq = jnp.sum(q * q, axis=1)
    ssq = jnp.sum(st * st, axis=0)
    dist1 = (qsq[None, :] - 2.0 * dot_t) + ssq[:, None]
    dist2 = (ssq[None, :] - 2.0 * dot) + qsq[:, None]

    td0, td1, ta0, ta1 = _tile_top2_axis0(dist1, bn)
    ta0 = ta0 + j * bn
    ta1 = ta1 + j * bn
    ad0 = jnp.where(j == 0, BIG, racc_ref[0, :])
    ad1 = jnp.where(j == 0, BIG, racc_ref[1, :])
    ai0 = jnp.where(j == 0, 0, ridx_acc_ref[0, :])
    ai1 = jnp.where(j == 0, 0, ridx_acc_ref[1, :])
    d0, d1, i0, i1 = _merge_top2(ad0, ad1, ai0, ai1, td0, td1, ta0, ta1)
    racc_ref[0, :] = d0
    racc_ref[1, :] = d1
    ridx_acc_ref[0, :] = i0
    ridx_acc_ref[1, :] = i1

    @pl.when(j == nj - 1)
    def _finalize_rows():
        ridx_ref[0, :] = i0
        ridx_ref[1, :] = i1

    td0c, td1c, ta0c, ta1c = _tile_top2_axis0(dist2, bm)
    ta0c = ta0c + i * bm
    ta1c = ta1c + i * bm
    jc = pl.ds(j * bn, bn)
    ad0c = jnp.where(i == 0, BIG, cacc_ref[0, jc])
    ad1c = jnp.where(i == 0, BIG, cacc_ref[1, jc])
    ai0c = jnp.where(i == 0, 0, cidx_acc_ref[0, jc])
    ai1c = jnp.where(i == 0, 0, cidx_acc_ref[1, jc])
    d0c, d1c, i0c, i1c = _merge_top2(ad0c, ad1c, ai0c, ai1c,
                                     td0c, td1c, ta0c, ta1c)
    cacc_ref[0, jc] = d0c
    cacc_ref[1, jc] = d1c
    cidx_acc_ref[0, jc] = i0c
    cidx_acc_ref[1, jc] = i1c

    @pl.when(i == ni - 1)
    def _finalize_cols():
        cidx_ref[0, jc] = i0c
        cidx_ref[1, jc] = i1c


def _stage1(q_feats, st_feats, bm=256, bn=256):
    n, c = q_feats.shape
    m = st_feats.shape[1]
    ni, nj = n // bm, m // bn
    body = functools.partial(_stage1_body, bm=bm, bn=bn, nj=nj, ni=ni, m=m)
    ridx, cidx = pl.pallas_call(
        body,
        grid=(ni, nj),
        in_specs=[
            pl.BlockSpec((bm, c), lambda i, j: (i, 0)),
            pl.BlockSpec((c, bn), lambda i, j: (0, j)),
        ],
        out_specs=[
            pl.BlockSpec((2, bm), lambda i, j: (0, i)),
            pl.BlockSpec((2, m), lambda i, j: (0, 0)),
        ],
        out_shape=[
            jax.ShapeDtypeStruct((2, n), jnp.int32),
            jax.ShapeDtypeStruct((2, m), jnp.int32),
        ],
        scratch_shapes=[
            pltpu.VMEM((2, bm), jnp.float32),
            pltpu.VMEM((2, bm), jnp.int32),
            pltpu.VMEM((2, m), jnp.float32),
            pltpu.VMEM((2, m), jnp.int32),
        ],
    )(q_feats, st_feats)
    return ridx.T, cidx.T


def _select_side(knn_indices, q_points, s_points, q_feats, s_feats):
    knn_feats = jnp.take(s_feats, knn_indices, axis=0)
    knn_similarities = 1.0 - jnp.sum(
        knn_feats * q_feats[:, None, :], axis=-1)
    weights = 1.0 - knn_similarities[:, 0] / (knn_similarities[:, 1] + EPS)
    _, q_corr = jax.lax.top_k(weights, NUM_CORR)
    s_corr = knn_indices[q_corr, 0]
    return (q_points[q_corr], s_points[s_corr], q_feats[q_corr],
            s_feats[s_corr], weights[q_corr])


def kernel(src_points, tgt_points, src_feats, tgt_feats):
    st = tgt_feats.T
    ridx, cidx = _stage1(src_feats, st)
    (sp1, tp1, sf1, tf1, w1) = _select_side(
        ridx, src_points, tgt_points, src_feats, tgt_feats)
    (tp2, sp2, tf2, sf2, w2) = _select_side(
        cidx, tgt_points, src_points, tgt_feats, src_feats)
    src_corr_points = jnp.concatenate([sp1, sp2], axis=0)
    tgt_corr_points = jnp.concatenate([tp1, tp2], axis=0)
    src_corr_feats = jnp.concatenate([sf1, sf2], axis=0)
    tgt_corr_feats = jnp.concatenate([tf1, tf2], axis=0)
    corr_weights = jnp.concatenate([w1, w2], axis=0)
    return (src_corr_points, tgt_corr_points, src_corr_feats,
            tgt_corr_feats, corr_weights)


# speedup vs baseline: 51.4038x; 51.4038x over previous
import functools

import jax
import jax.numpy as jnp
from jax import lax
from jax.experimental import pallas as pl
from jax.experimental.pallas import tpu as pltpu

NUM_CORR = 256
EPS = 1e-08
BIG = 3.0e38


def _tile_top2_axis0(dist, nrow):
    row = lax.broadcasted_iota(jnp.int32, dist.shape, 0)
    d0 = jnp.min(dist, axis=0)
    a0 = jnp.min(jnp.where(dist == d0[None, :], row, nrow), axis=0)
    dist_m = jnp.where(row == a0[None, :], BIG, dist)
    d1 = jnp.min(dist_m, axis=0)
    a1 = jnp.min(jnp.where(dist_m == d1[None, :], row, nrow), axis=0)
    return d0, d1, a0, a1


def _merge_top2(ad0, ad1, ai0, ai1, td0, td1, ti0, ti1):
    a_first = ad0 <= td0
    d0 = jnp.where(a_first, ad0, td0)
    i0 = jnp.where(a_first, ai0, ti0)
    loser_d = jnp.where(a_first, td0, ad0)
    loser_i = jnp.where(a_first, ti0, ai0)
    inner_a = ad1 <= td1
    inner_d = jnp.where(inner_a, ad1, td1)
    inner_i = jnp.where(inner_a, ai1, ti1)
    take_loser = loser_d <= inner_d
    d1 = jnp.where(take_loser, loser_d, inner_d)
    i1 = jnp.where(take_loser, loser_i, inner_i)
    return d0, d1, i0, i1


def _stage1_body(q_ref, st_ref, qsq_row_ref, ssq_row_ref,
                 qsq_col_ref, ssq_col_ref, ridx_ref, cidx_ref,
                 racc_ref, ridx_acc_ref, cacc_ref, cidx_acc_ref,
                 *, bm, bn, nj, ni, m):
    i = pl.program_id(0)
    j = pl.program_id(1)
    q = q_ref[...]
    st = st_ref[...]
    dot = jax.lax.dot_general(q, st, (((1,), (0,)), ((), ())),
                              preferred_element_type=jnp.float32)
    dot_t = jax.lax.dot_general(st, q, (((0,), (1,)), ((), ())),
                                preferred_element_type=jnp.float32)
    qsq_row = qsq_row_ref[...]
    ssq_row = ssq_row_ref[...]
    qsq_col = qsq_col_ref[...]
    ssq_col = ssq_col_ref[...]
    dist1 = (qsq_row - 2.0 * dot_t) + ssq_col
    dist2 = (ssq_row - 2.0 * dot) + qsq_col

    td0, td1, ta0, ta1 = _tile_top2_axis0(dist1, bn)
    ta0 = ta0 + j * bn
    ta1 = ta1 + j * bn
    ad0 = jnp.where(j == 0, BIG, racc_ref[0, :])
    ad1 = jnp.where(j == 0, BIG, racc_ref[1, :])
    ai0 = jnp.where(j == 0, 0, ridx_acc_ref[0, :])
    ai1 = jnp.where(j == 0, 0, ridx_acc_ref[1, :])
    d0, d1, i0, i1 = _merge_top2(ad0, ad1, ai0, ai1, td0, td1, ta0, ta1)
    racc_ref[0, :] = d0
    racc_ref[1, :] = d1
    ridx_acc_ref[0, :] = i0
    ridx_acc_ref[1, :] = i1

    @pl.when(j == nj - 1)
    def _finalize_rows():
        ridx_ref[0, :] = i0
        ridx_ref[1, :] = i1

    td0c, td1c, ta0c, ta1c = _tile_top2_axis0(dist2, bm)
    ta0c = ta0c + i * bm
    ta1c = ta1c + i * bm
    jc = pl.ds(pl.multiple_of(j * bn, bn), bn)
    ad0c = jnp.where(i == 0, BIG, cacc_ref[0, jc])
    ad1c = jnp.where(i == 0, BIG, cacc_ref[1, jc])
    ai0c = jnp.where(i == 0, 0, cidx_acc_ref[0, jc])
    ai1c = jnp.where(i == 0, 0, cidx_acc_ref[1, jc])
    d0c, d1c, i0c, i1c = _merge_top2(ad0c, ad1c, ai0c, ai1c,
                                     td0c, td1c, ta0c, ta1c)
    cacc_ref[0, jc] = d0c
    cacc_ref[1, jc] = d1c
    cidx_acc_ref[0, jc] = i0c
    cidx_acc_ref[1, jc] = i1c

    @pl.when(i == ni - 1)
    def _finalize_cols():
        cidx_ref[0, jc] = i0c
        cidx_ref[1, jc] = i1c


def _stage1(q_feats, st_feats, qsq, ssq, bm=512, bn=512):
    n, c = q_feats.shape
    m = st_feats.shape[1]
    ni, nj = n // bm, m // bn
    body = functools.partial(_stage1_body, bm=bm, bn=bn, nj=nj, ni=ni, m=m)
    ridx, cidx = pl.pallas_call(
        body,
        grid=(ni, nj),
        in_specs=[
            pl.BlockSpec((bm, c), lambda i, j: (i, 0)),
            pl.BlockSpec((c, bn), lambda i, j: (0, j)),
            pl.BlockSpec((1, bm), lambda i, j: (0, i)),
            pl.BlockSpec((1, bn), lambda i, j: (0, j)),
            pl.BlockSpec((bm, 1), lambda i, j: (i, 0)),
            pl.BlockSpec((bn, 1), lambda i, j: (j, 0)),
        ],
        out_specs=[
            pl.BlockSpec((2, bm), lambda i, j: (0, i)),
            pl.BlockSpec((2, m), lambda i, j: (0, 0)),
        ],
        out_shape=[
            jax.ShapeDtypeStruct((2, n), jnp.int32),
            jax.ShapeDtypeStruct((2, m), jnp.int32),
        ],
        scratch_shapes=[
            pltpu.VMEM((2, bm), jnp.float32),
            pltpu.VMEM((2, bm), jnp.int32),
            pltpu.VMEM((2, m), jnp.float32),
            pltpu.VMEM((2, m), jnp.int32),
        ],
    )(q_feats, st_feats, qsq[None, :], ssq[None, :],
      qsq[:, None], ssq[:, None])
    return ridx.T, cidx.T


def _select_side(knn_indices, q_points, s_points, q_feats, s_feats):
    knn_feats = jnp.take(s_feats, knn_indices, axis=0)
    knn_similarities = 1.0 - jnp.sum(
        knn_feats * q_feats[:, None, :], axis=-1)
    weights = 1.0 - knn_similarities[:, 0] / (knn_similarities[:, 1] + EPS)
    _, q_corr = jax.lax.top_k(weights, NUM_CORR)
    s_corr = knn_indices[q_corr, 0]
    return (q_points[q_corr], s_points[s_corr], q_feats[q_corr],
            s_feats[s_corr], weights[q_corr])


def kernel(src_points, tgt_points, src_feats, tgt_feats):
    st = tgt_feats.T
    qsq = jnp.sum(src_feats ** 2, axis=1)
    ssq = jnp.sum(tgt_feats ** 2, axis=1)
    ridx, cidx = _stage1(src_feats, st, qsq, ssq)
    (sp1, tp1, sf1, tf1, w1) = _select_side(
        ridx, src_points, tgt_points, src_feats, tgt_feats)
    (tp2, sp2, tf2, sf2, w2) = _select_side(
        cidx, tgt_points, src_points, tgt_feats, src_feats)
    src_corr_points = jnp.concatenate([sp1, sp2], axis=0)
    tgt_corr_points = jnp.concatenate([tp1, tp2], axis=0)
    src_corr_feats = jnp.concatenate([sf1, sf2], axis=0)
    tgt_corr_feats = jnp.concatenate([tf1, tf2], axis=0)
    corr_weights = jnp.concatenate([w1, w2], axis=0)
    return (src_corr_points, tgt_corr_points, src_corr_feats,
            tgt_corr_feats, corr_weights)


# speedup vs baseline: 57.7307x; 1.1231x over previous
import functools

import jax
import jax.numpy as jnp
from jax import lax
from jax.experimental import pallas as pl
from jax.experimental.pallas import tpu as pltpu

NUM_CORR = 256
EPS = 1e-08
BIG = 3.0e38


def _tile_top2_axis0(dist, row, nrow):
    fnrow = float(nrow)
    d0 = jnp.min(dist, axis=0)
    a0 = jnp.min(jnp.where(dist == d0[None, :], row, fnrow), axis=0)
    dist_m = jnp.where(row == a0[None, :], BIG, dist)
    d1 = jnp.min(dist_m, axis=0)
    a1 = jnp.min(jnp.where(dist_m == d1[None, :], row, fnrow), axis=0)
    return d0, d1, a0, a1


def _merge_top2(ad0, ad1, ai0, ai1, td0, td1, ti0, ti1):
    a_first = ad0 <= td0
    d0 = jnp.where(a_first, ad0, td0)
    i0 = jnp.where(a_first, ai0, ti0)
    loser_d = jnp.where(a_first, td0, ad0)
    loser_i = jnp.where(a_first, ti0, ai0)
    inner_a = ad1 <= td1
    inner_d = jnp.where(inner_a, ad1, td1)
    inner_i = jnp.where(inner_a, ai1, ti1)
    take_loser = loser_d <= inner_d
    d1 = jnp.where(take_loser, loser_d, inner_d)
    i1 = jnp.where(take_loser, loser_i, inner_i)
    return d0, d1, i0, i1


def _stage1_body(qn_ref, st_ref, qsq_row_ref, ssq_row_ref,
                 qsq_col_ref, ssq_col_ref, row1_ref, row2_ref,
                 ridx_ref, cidx_ref,
                 racc_ref, ridx_acc_ref, cacc_ref, cidx_acc_ref,
                 *, bm, bn, nj, ni, m):
    i = pl.program_id(0)
    j = pl.program_id(1)
    qn = qn_ref[...]
    st = st_ref[...]
    dotn = jax.lax.dot_general(qn, st, (((1,), (0,)), ((), ())),
                               preferred_element_type=jnp.float32)
    dotn_t = jax.lax.dot_general(st, qn, (((0,), (1,)), ((), ())),
                                 preferred_element_type=jnp.float32)
    qsq_row = qsq_row_ref[...]
    ssq_row = ssq_row_ref[...]
    qsq_col = qsq_col_ref[...]
    ssq_col = ssq_col_ref[...]
    dist1 = (qsq_row + dotn_t) + ssq_col
    dist2 = (ssq_row + dotn) + qsq_col

    td0, td1, ta0, ta1 = _tile_top2_axis0(dist1, row1_ref[...], bn)
    ta0 = ta0 + j * bn
    ta1 = ta1 + j * bn
    ad0 = jnp.where(j == 0, BIG, racc_ref[0, :])
    ad1 = jnp.where(j == 0, BIG, racc_ref[1, :])
    ai0 = jnp.where(j == 0, 0.0, ridx_acc_ref[0, :])
    ai1 = jnp.where(j == 0, 0.0, ridx_acc_ref[1, :])
    d0, d1, i0, i1 = _merge_top2(ad0, ad1, ai0, ai1, td0, td1, ta0, ta1)
    racc_ref[0, :] = d0
    racc_ref[1, :] = d1
    ridx_acc_ref[0, :] = i0
    ridx_acc_ref[1, :] = i1

    @pl.when(j == nj - 1)
    def _finalize_rows():
        ridx_ref[0, :] = i0.astype(jnp.int32)
        ridx_ref[1, :] = i1.astype(jnp.int32)

    td0c, td1c, ta0c, ta1c = _tile_top2_axis0(dist2, row2_ref[...], bm)
    ta0c = ta0c + i * bm
    ta1c = ta1c + i * bm
    jc = pl.ds(pl.multiple_of(j * bn, bn), bn)
    ad0c = jnp.where(i == 0, BIG, cacc_ref[0, jc])
    ad1c = jnp.where(i == 0, BIG, cacc_ref[1, jc])
    ai0c = jnp.where(i == 0, 0.0, cidx_acc_ref[0, jc])
    ai1c = jnp.where(i == 0, 0.0, cidx_acc_ref[1, jc])
    d0c, d1c, i0c, i1c = _merge_top2(ad0c, ad1c, ai0c, ai1c,
                                     td0c, td1c, ta0c, ta1c)
    cacc_ref[0, jc] = d0c
    cacc_ref[1, jc] = d1c
    cidx_acc_ref[0, jc] = i0c
    cidx_acc_ref[1, jc] = i1c

    @pl.when(i == ni - 1)
    def _finalize_cols():
        cidx_ref[0, jc] = i0c.astype(jnp.int32)
        cidx_ref[1, jc] = i1c.astype(jnp.int32)


def _stage1(qn_feats, st_feats, qsq, ssq, bm=512, bn=1024):
    n, c = qn_feats.shape
    m = st_feats.shape[1]
    ni, nj = n // bm, m // bn
    body = functools.partial(_stage1_body, bm=bm, bn=bn, nj=nj, ni=ni, m=m)
    ridx, cidx = pl.pallas_call(
        body,
        grid=(ni, nj),
        in_specs=[
            pl.BlockSpec((bm, c), lambda i, j: (i, 0)),
            pl.BlockSpec((c, bn), lambda i, j: (0, j)),
            pl.BlockSpec((1, bm), lambda i, j: (0, i)),
            pl.BlockSpec((1, bn), lambda i, j: (0, j)),
            pl.BlockSpec((bm, 1), lambda i, j: (i, 0)),
            pl.BlockSpec((bn, 1), lambda i, j: (j, 0)),
            pl.BlockSpec((bn, 1), lambda i, j: (0, 0)),
            pl.BlockSpec((bm, 1), lambda i, j: (0, 0)),
        ],
        out_specs=[
            pl.BlockSpec((2, bm), lambda i, j: (0, i)),
            pl.BlockSpec((2, m), lambda i, j: (0, 0)),
        ],
        out_shape=[
            jax.ShapeDtypeStruct((2, n), jnp.int32),
            jax.ShapeDtypeStruct((2, m), jnp.int32),
        ],
        scratch_shapes=[
            pltpu.VMEM((2, bm), jnp.float32),
            pltpu.VMEM((2, bm), jnp.float32),
            pltpu.VMEM((2, m), jnp.float32),
            pltpu.VMEM((2, m), jnp.float32),
        ],
    )(qn_feats, st_feats, qsq[None, :], ssq[None, :],
      qsq[:, None], ssq[:, None],
      jnp.arange(bn, dtype=jnp.float32)[:, None],
      jnp.arange(bm, dtype=jnp.float32)[:, None])
    return ridx.T, cidx.T


def _select_side(knn_indices, q_points, s_points, q_feats, s_feats):
    knn_feats = jnp.take(s_feats, knn_indices, axis=0)
    knn_similarities = 1.0 - jnp.sum(
        knn_feats * q_feats[:, None, :], axis=-1)
    weights = 1.0 - knn_similarities[:, 0] / (knn_similarities[:, 1] + EPS)
    _, q_corr = jax.lax.top_k(weights, NUM_CORR)
    s_corr = knn_indices[q_corr, 0]
    return (q_points[q_corr], s_points[s_corr], q_feats[q_corr],
            s_feats[s_corr], weights[q_corr])


def kernel(src_points, tgt_points, src_feats, tgt_feats):
    st = tgt_feats.T
    qn = src_feats * -2.0
    qsq = jnp.sum(src_feats ** 2, axis=1)
    ssq = jnp.sum(tgt_feats ** 2, axis=1)
    ridx, cidx = _stage1(qn, st, qsq, ssq)
    (sp1, tp1, sf1, tf1, w1) = _select_side(
        ridx, src_points, tgt_points, src_feats, tgt_feats)
    (tp2, sp2, tf2, sf2, w2) = _select_side(
        cidx, tgt_points, src_points, tgt_feats, src_feats)
    src_corr_points = jnp.concatenate([sp1, sp2], axis=0)
    tgt_corr_points = jnp.concatenate([tp1, tp2], axis=0)
    src_corr_feats = jnp.concatenate([sf1, sf2], axis=0)
    tgt_corr_feats = jnp.concatenate([tf1, tf2], axis=0)
    corr_weights = jnp.concatenate([w1, w2], axis=0)
    return (src_corr_points, tgt_corr_points, src_corr_feats,
            tgt_corr_feats, corr_weights)


# speedup vs baseline: 58.8673x; 1.0197x over previous
import functools

import jax
import jax.numpy as jnp
from jax import lax
from jax.experimental import pallas as pl
from jax.experimental.pallas import tpu as pltpu
from jax.experimental.pallas import tpu_sc as plsc

_SC_NC = 2
_SC_NS = 16
_SC_NW = _SC_NC * _SC_NS

NUM_CORR = 256
EPS = 1e-08
BIG = 3.0e38


def _tile_top2_axis0(dist, row, nrow):
    fnrow = float(nrow)
    d0 = jnp.min(dist, axis=0)
    a0 = jnp.min(jnp.where(dist == d0[None, :], row, fnrow), axis=0)
    dist_m = jnp.where(row == a0[None, :], BIG, dist)
    d1 = jnp.min(dist_m, axis=0)
    a1 = jnp.min(jnp.where(dist_m == d1[None, :], row, fnrow), axis=0)
    return d0, d1, a0, a1


def _merge_top2(ad0, ad1, ai0, ai1, td0, td1, ti0, ti1):
    a_first = ad0 <= td0
    d0 = jnp.where(a_first, ad0, td0)
    i0 = jnp.where(a_first, ai0, ti0)
    loser_d = jnp.where(a_first, td0, ad0)
    loser_i = jnp.where(a_first, ti0, ai0)
    inner_a = ad1 <= td1
    inner_d = jnp.where(inner_a, ad1, td1)
    inner_i = jnp.where(inner_a, ai1, ti1)
    take_loser = loser_d <= inner_d
    d1 = jnp.where(take_loser, loser_d, inner_d)
    i1 = jnp.where(take_loser, loser_i, inner_i)
    return d0, d1, i0, i1


def _stage1_body(qn_ref, st_ref, qsq_row_ref, ssq_row_ref,
                 qsq_col_ref, ssq_col_ref, row1_ref, row2_ref,
                 ridx_ref, cidx_ref,
                 racc_ref, ridx_acc_ref, cacc_ref, cidx_acc_ref,
                 *, bm, bn, nj, ni, m):
    i = pl.program_id(0)
    j = pl.program_id(1)
    qn = qn_ref[...]
    st = st_ref[...]
    dotn = jax.lax.dot_general(qn, st, (((1,), (0,)), ((), ())),
                               preferred_element_type=jnp.float32)
    dotn_t = jax.lax.dot_general(st, qn, (((0,), (1,)), ((), ())),
                                 preferred_element_type=jnp.float32)
    qsq_row = qsq_row_ref[...]
    ssq_row = ssq_row_ref[...]
    qsq_col = qsq_col_ref[...]
    ssq_col = ssq_col_ref[...]
    dist1 = (qsq_row + dotn_t) + ssq_col
    dist2 = (ssq_row + dotn) + qsq_col

    td0, td1, ta0, ta1 = _tile_top2_axis0(dist1, row1_ref[...], bn)
    ta0 = ta0 + j * bn
    ta1 = ta1 + j * bn
    ad0 = jnp.where(j == 0, BIG, racc_ref[0, :])
    ad1 = jnp.where(j == 0, BIG, racc_ref[1, :])
    ai0 = jnp.where(j == 0, 0.0, ridx_acc_ref[0, :])
    ai1 = jnp.where(j == 0, 0.0, ridx_acc_ref[1, :])
    d0, d1, i0, i1 = _merge_top2(ad0, ad1, ai0, ai1, td0, td1, ta0, ta1)
    racc_ref[0, :] = d0
    racc_ref[1, :] = d1
    ridx_acc_ref[0, :] = i0
    ridx_acc_ref[1, :] = i1

    @pl.when(j == nj - 1)
    def _finalize_rows():
        ridx_ref[0, :] = i0.astype(jnp.int32)
        ridx_ref[1, :] = i1.astype(jnp.int32)

    td0c, td1c, ta0c, ta1c = _tile_top2_axis0(dist2, row2_ref[...], bm)
    ta0c = ta0c + i * bm
    ta1c = ta1c + i * bm
    jc = pl.ds(pl.multiple_of(j * bn, bn), bn)
    ad0c = jnp.where(i == 0, BIG, cacc_ref[0, jc])
    ad1c = jnp.where(i == 0, BIG, cacc_ref[1, jc])
    ai0c = jnp.where(i == 0, 0.0, cidx_acc_ref[0, jc])
    ai1c = jnp.where(i == 0, 0.0, cidx_acc_ref[1, jc])
    d0c, d1c, i0c, i1c = _merge_top2(ad0c, ad1c, ai0c, ai1c,
                                     td0c, td1c, ta0c, ta1c)
    cacc_ref[0, jc] = d0c
    cacc_ref[1, jc] = d1c
    cidx_acc_ref[0, jc] = i0c
    cidx_acc_ref[1, jc] = i1c

    @pl.when(i == ni - 1)
    def _finalize_cols():
        cidx_ref[0, jc] = i0c.astype(jnp.int32)
        cidx_ref[1, jc] = i1c.astype(jnp.int32)


def _stage1(qn_feats, st_feats, qsq, ssq, bm=512, bn=1024):
    n, c = qn_feats.shape
    m = st_feats.shape[1]
    ni, nj = n // bm, m // bn
    body = functools.partial(_stage1_body, bm=bm, bn=bn, nj=nj, ni=ni, m=m)
    ridx, cidx = pl.pallas_call(
        body,
        grid=(ni, nj),
        in_specs=[
            pl.BlockSpec((bm, c), lambda i, j: (i, 0)),
            pl.BlockSpec((c, bn), lambda i, j: (0, j)),
            pl.BlockSpec((1, bm), lambda i, j: (0, i)),
            pl.BlockSpec((1, bn), lambda i, j: (0, j)),
            pl.BlockSpec((bm, 1), lambda i, j: (i, 0)),
            pl.BlockSpec((bn, 1), lambda i, j: (j, 0)),
            pl.BlockSpec((bn, 1), lambda i, j: (0, 0)),
            pl.BlockSpec((bm, 1), lambda i, j: (0, 0)),
        ],
        out_specs=[
            pl.BlockSpec((2, bm), lambda i, j: (0, i)),
            pl.BlockSpec((2, m), lambda i, j: (0, 0)),
        ],
        out_shape=[
            jax.ShapeDtypeStruct((2, n), jnp.int32),
            jax.ShapeDtypeStruct((2, m), jnp.int32),
        ],
        scratch_shapes=[
            pltpu.VMEM((2, bm), jnp.float32),
            pltpu.VMEM((2, bm), jnp.float32),
            pltpu.VMEM((2, m), jnp.float32),
            pltpu.VMEM((2, m), jnp.float32),
        ],
    )(qn_feats, st_feats, qsq[None, :], ssq[None, :],
      qsq[:, None], ssq[:, None],
      jnp.arange(bn, dtype=jnp.float32)[:, None],
      jnp.arange(bm, dtype=jnp.float32)[:, None])
    return ridx.T, cidx.T


def _gather_corr_feats_sc(src_feats, tgt_feats, idx_src, idx_tgt):
    b, d = idx_src.shape[0], src_feats.shape[1]
    bw = b // _SC_NW
    mesh = plsc.VectorSubcoreMesh(core_axis_name="c", subcore_axis_name="s",
                                  num_cores=_SC_NC, num_subcores=_SC_NS)

    @functools.partial(
        pl.kernel, mesh=mesh,
        out_type=[jax.ShapeDtypeStruct((b, d), jnp.float32),
                  jax.ShapeDtypeStruct((b, d), jnp.float32)],
        scratch_types=[
            pltpu.VMEM((bw,), jnp.int32),
            pltpu.VMEM((bw,), jnp.int32),
            pltpu.VMEM((bw, d), jnp.float32),
            pltpu.VMEM((bw, d), jnp.float32),
            pltpu.SemaphoreType.DMA,
        ],
    )
    def k(sfeat_hbm, tfeat_hbm, isrc_hbm, itgt_hbm, osrc_hbm, otgt_hbm,
          isrc_v, itgt_v, srows_v, trows_v, sem):
        wid = lax.axis_index("s") * _SC_NC + lax.axis_index("c")
        base = wid * bw
        pltpu.sync_copy(isrc_hbm.at[pl.ds(base, bw)], isrc_v)
        pltpu.sync_copy(itgt_hbm.at[pl.ds(base, bw)], itgt_v)
        cp_s = pltpu.async_copy(sfeat_hbm.at[isrc_v], srows_v, sem)
        cp_t = pltpu.async_copy(tfeat_hbm.at[itgt_v], trows_v, sem)
        cp_s.wait()
        cp_t.wait()
        pltpu.sync_copy(srows_v, osrc_hbm.at[pl.ds(base, bw)])
        pltpu.sync_copy(trows_v, otgt_hbm.at[pl.ds(base, bw)])

    return k(src_feats, tgt_feats, idx_src, idx_tgt)


def _select_side(knn_indices, q_feats, s_feats):
    knn_feats = jnp.take(s_feats, knn_indices, axis=0)
    knn_similarities = 1.0 - jnp.sum(
        knn_feats * q_feats[:, None, :], axis=-1)
    weights = 1.0 - knn_similarities[:, 0] / (knn_similarities[:, 1] + EPS)
    _, q_corr = jax.lax.top_k(weights, NUM_CORR)
    s_corr = knn_indices[q_corr, 0]
    return q_corr, s_corr, weights[q_corr]


def kernel(src_points, tgt_points, src_feats, tgt_feats):
    st = tgt_feats.T
    qn = src_feats * -2.0
    qsq = jnp.sum(src_feats ** 2, axis=1)
    ssq = jnp.sum(tgt_feats ** 2, axis=1)
    ridx, cidx = _stage1(qn, st, qsq, ssq)
    q1, s1, w1 = _select_side(ridx, src_feats, tgt_feats)
    q2, s2, w2 = _select_side(cidx, tgt_feats, src_feats)
    idx_src = jnp.concatenate([q1, s2], axis=0)
    idx_tgt = jnp.concatenate([s1, q2], axis=0)
    src_corr_feats, tgt_corr_feats = _gather_corr_feats_sc(
        src_feats, tgt_feats, idx_src, idx_tgt)
    src_corr_points = src_points[idx_src]
    tgt_corr_points = tgt_points[idx_tgt]
    corr_weights = jnp.concatenate([w1, w2], axis=0)
    return (src_corr_points, tgt_corr_points, src_corr_feats,
            tgt_corr_feats, corr_weights)
